# baseline (speedup 1.0000x reference)
import functools

import jax
import jax.numpy as jnp
from jax import lax
from jax.experimental import pallas as pl
from jax.experimental.pallas import tpu as pltpu
from jax.experimental.pallas import tpu_sc as plsc

N = 10000
D = 128
E = 640000
K = 6

NW = 32
NS = 16
CHUNK = 128
NB = 2
IB = 16
OUTER = 10
CPT = IB * OUTER
EPT = CPT * CHUNK
E_PAD = NW * EPT
N_PAD = 10008
DEG_PAD = 10016
RPT = 624
ZREM = N_PAD - NS * RPT
OREM = N - NS * RPT

_mesh = plsc.VectorSubcoreMesh(core_axis_name="c", subcore_axis_name="s")


def _sc_deg_body(col_hbm, out_hbm, colv, degv, sem):
    cid = lax.axis_index("c")
    sid = lax.axis_index("s")
    wid = cid * NS + sid
    pltpu.async_copy(col_hbm.at[wid], colv, sem).wait()
    zeros16 = jnp.zeros((16,), jnp.float32)

    def zbody(i, carry):
        degv[pl.ds(i * 16, 16)] = zeros16
        return carry

    lax.fori_loop(0, DEG_PAD // 16, zbody, 0)
    ones16 = jnp.ones((16,), jnp.float32)

    def ebody(j, carry):
        idx = colv[pl.ds(j * 16, 16)]
        plsc.addupdate_scatter(degv, [idx], ones16)
        return carry

    lax.fori_loop(0, EPT // 16, ebody, 0)
    pltpu.sync_copy(degv, out_hbm.at[wid])


_sc_deg = pl.kernel(
    _sc_deg_body,
    out_type=jax.ShapeDtypeStruct((NW, DEG_PAD), jnp.float32),
    mesh=_mesh,
    compiler_params=pltpu.CompilerParams(needs_layout_passes=False),
    scratch_types=[
        pltpu.VMEM((EPT,), jnp.int32),
        pltpu.VMEM((DEG_PAD,), jnp.float32),
        pltpu.SemaphoreType.DMA,
    ],
)


def _sc_hop_body(g_hbm, idx_hbm, zer_hbm, out_hbm,
                 idxb, rows0, rows1, acc,
                 gsem0, gsem1, ssem0, ssem1):
    cid = lax.axis_index("c")
    sid = lax.axis_index("s")
    wid = cid * NS + sid
    rows = (rows0, rows1)
    gsem = (gsem0, gsem1)
    ssem = (ssem0, ssem1)

    def _wait(b, sem):
        pltpu.make_async_copy(zer_hbm.at[pl.ds(0, CHUNK)], rows[b], sem[b]).wait()

    def _gather(j, b):
        pltpu.async_copy(g_hbm.at[cid].at[idxb.at[0, j]], rows[b], gsem[b])

    def _scatter(j, b):
        pltpu.async_copy(rows[b], acc.at[idxb.at[1, j]], ssem[b], add=True)

    pltpu.sync_copy(zer_hbm.at[pl.ds(sid * RPT, RPT)], acc.at[pl.ds(sid * RPT, RPT)])

    @pl.when(sid == 0)
    def _():
        pltpu.sync_copy(zer_hbm.at[pl.ds(NS * RPT, ZREM)], acc.at[pl.ds(NS * RPT, ZREM)])

    plsc.subcore_barrier()

    def obody(o, carry):
        @pl.when(o > 0)
        def _():
            for b in range(NB):
                _wait(b, ssem)

        pltpu.sync_copy(idx_hbm.at[wid, :, pl.ds(o * IB, IB)], idxb)
        for b in range(NB):
            _gather(b, b)

        def ibody(p, c2):
            for b in range(NB):
                _wait(b, gsem)
                _scatter(NB * p + b, b)
            for b in range(NB):
                _wait(b, ssem)
                _gather(NB * p + NB + b, b)
            return c2

        lax.fori_loop(0, IB // NB - 1, ibody, 0)
        for b in range(NB):
            _wait(b, gsem)
            _scatter(IB - NB + b, b)
        return carry

    lax.fori_loop(0, OUTER, obody, 0)
    for b in range(NB):
        _wait(b, ssem)
    plsc.subcore_barrier()
    pltpu.sync_copy(acc.at[pl.ds(sid * RPT, RPT)], out_hbm.at[cid, pl.ds(sid * RPT, RPT)])

    @pl.when(sid == 0)
    def _():
        pltpu.sync_copy(acc.at[pl.ds(NS * RPT, OREM)], out_hbm.at[cid, pl.ds(NS * RPT, OREM)])


_sc_hop = pl.kernel(
    _sc_hop_body,
    out_type=jax.ShapeDtypeStruct((2, N, D), jnp.float32),
    mesh=_mesh,
    compiler_params=pltpu.CompilerParams(needs_layout_passes=False),
    scratch_types=[
        pltpu.VMEM((2, IB, CHUNK), jnp.int32),
        pltpu.VMEM((CHUNK, D), jnp.float32),
        pltpu.VMEM((CHUNK, D), jnp.float32),
        pltpu.VMEM_SHARED((N_PAD, D), jnp.float32),
        pltpu.SemaphoreType.DMA,
        pltpu.SemaphoreType.DMA,
        pltpu.SemaphoreType.DMA,
        pltpu.SemaphoreType.DMA,
    ],
)


def _gelu(x):
    return x * 0.5 * (1.0 + lax.erf(x * 0.7071067811865476))


def _tc_pre_body(x_ref, degp_ref, gamma_ref, beta_ref, w0_ref, b0_ref, wt0_ref,
                 xb_ref, g_ref, acc_ref, dis_ref):
    x = x_ref[...]
    mu = jnp.mean(x, axis=0, keepdims=True)
    xc = x - mu
    var = jnp.mean(xc * xc, axis=0, keepdims=True)
    xb = xc * lax.rsqrt(var + 1e-5) * gamma_ref[...] + beta_ref[...]
    xb_ref[...] = xb
    h = _gelu(jnp.dot(xb, w0_ref[...], preferred_element_type=jnp.float32) + b0_ref[...])
    deg = jnp.sum(degp_ref[...], axis=1, keepdims=True)[:N]
    dis = jnp.where(deg > 0, lax.rsqrt(jnp.maximum(deg, 1e-12)), 0.0)
    dis_ref[...] = dis
    g = dis * h
    g_ref[0] = g
    g_ref[1] = g
    acc_ref[...] = jnp.dot(h, wt0_ref[...], preferred_element_type=jnp.float32)


_tc_pre = pl.pallas_call(
    _tc_pre_body,
    out_shape=(
        jax.ShapeDtypeStruct((N, D), jnp.float32),
        jax.ShapeDtypeStruct((2, N, D), jnp.float32),
        jax.ShapeDtypeStruct((N, D), jnp.float32),
        jax.ShapeDtypeStruct((N, 1), jnp.float32),
    ),
)


def _tc_hop_body(s_ref, dis_ref, wt_ref, acc_in_ref, g_ref, acc_out_ref):
    s = s_ref[0] + s_ref[1]
    dis = dis_ref[...]
    cur = dis * s
    g = dis * cur
    g_ref[0] = g
    g_ref[1] = g
    acc_out_ref[...] = acc_in_ref[...] + jnp.dot(
        cur, wt_ref[...], preferred_element_type=jnp.float32)


_tc_hop = pl.pallas_call(
    _tc_hop_body,
    out_shape=(
        jax.ShapeDtypeStruct((2, N, D), jnp.float32),
        jax.ShapeDtypeStruct((N, D), jnp.float32),
    ),
)


def _tc_post_body(xb_ref, acc_ref, bt_ref, w1a_ref, w1b_ref, b1_ref,
                  w2_ref, b2_ref, wc_ref, bc_ref, we_ref, be_ref,
                  a_ref, cl_ref):
    h = _gelu(acc_ref[...] + bt_ref[...])
    a1 = _gelu(jnp.dot(xb_ref[...], w1a_ref[...], preferred_element_type=jnp.float32)
               + jnp.dot(h, w1b_ref[...], preferred_element_type=jnp.float32)
               + b1_ref[...])
    a2 = _gelu(jnp.dot(a1, w2_ref[...], preferred_element_type=jnp.float32) + b2_ref[...])
    a_ref[...] = a2
    cl = _gelu(jnp.dot(a2, wc_ref[...], preferred_element_type=jnp.float32) + bc_ref[...])
    logits = jnp.dot(cl, we_ref[...], preferred_element_type=jnp.float32) + be_ref[...]
    m = jnp.max(logits, axis=1, keepdims=True)
    e = jnp.exp(logits - m)
    cl_ref[...] = e / jnp.sum(e, axis=1, keepdims=True)


_tc_post = pl.pallas_call(
    _tc_post_body,
    out_shape=(
        jax.ShapeDtypeStruct((N, 64), jnp.float32),
        jax.ShapeDtypeStruct((N, 16), jnp.float32),
    ),
)


def kernel(x, edge_index, gamma, beta, W0, b0, Wt, bt, W1, b1, W2, b2, Wc, bc, We, be):
    row = edge_index[0]
    col = edge_index[1]
    pad = E_PAD - E
    rowp = jnp.concatenate([row, jnp.zeros((pad,), jnp.int32)]).reshape(NW, CPT, CHUNK)
    colp = jnp.concatenate([col, jnp.full((pad,), N, jnp.int32)]).reshape(NW, CPT, CHUNK)
    idxp = jnp.stack([rowp, colp], axis=1)
    degp = _sc_deg(colp.reshape(NW, EPT))
    zer = jnp.zeros((N_PAD, D), jnp.float32)
    xb, g, acc, dis = _tc_pre(x, degp.T, gamma.reshape(1, D), beta.reshape(1, D),
                              W0, b0.reshape(1, D), Wt[0])
    for k in range(1, K + 1):
        s = _sc_hop(g, idxp, zer)
        g, acc = _tc_hop(s, dis, Wt[k], acc)
    a, cl = _tc_post(xb, acc, bt.reshape(1, D), W1[:D], W1[D:], b1.reshape(1, 128),
                     W2, b2.reshape(1, 64), Wc, bc.reshape(1, 64),
                     We, be.reshape(1, 16))
    return (a, cl)


# speedup vs baseline: 1.1282x; 1.1282x over previous
import functools

import jax
import jax.numpy as jnp
from jax import lax
from jax.experimental import pallas as pl
from jax.experimental.pallas import tpu as pltpu
from jax.experimental.pallas import tpu_sc as plsc

N = 10000
D = 128
E = 640000
K = 6

NW = 32
NS = 16
CHUNK = 128
NB = 2
IB = 32
OUTER = 5
CPT = IB * OUTER
EPT = CPT * CHUNK
E_PAD = NW * EPT
N_PAD = 10008
DEG_PAD = 10016
RPT = 624
ZREM = N_PAD - NS * RPT
OREM = N - NS * RPT

_mesh = plsc.VectorSubcoreMesh(core_axis_name="c", subcore_axis_name="s")


def _sc_deg_body(col_hbm, out_hbm, colv, degv, sem):
    cid = lax.axis_index("c")
    sid = lax.axis_index("s")
    wid = cid * NS + sid
    pltpu.async_copy(col_hbm.at[wid], colv, sem).wait()
    zeros16 = jnp.zeros((16,), jnp.float32)

    def zbody(i, carry):
        degv[pl.ds(i * 16, 16)] = zeros16
        return carry

    lax.fori_loop(0, DEG_PAD // 16, zbody, 0)
    ones16 = jnp.ones((16,), jnp.float32)

    def ebody(j, carry):
        idx = colv[pl.ds(j * 16, 16)]
        plsc.addupdate_scatter(degv, [idx], ones16)
        return carry

    lax.fori_loop(0, EPT // 16, ebody, 0)
    pltpu.sync_copy(degv, out_hbm.at[wid])


_sc_deg = pl.kernel(
    _sc_deg_body,
    out_type=jax.ShapeDtypeStruct((NW, DEG_PAD), jnp.float32),
    mesh=_mesh,
    compiler_params=pltpu.CompilerParams(needs_layout_passes=False),
    scratch_types=[
        pltpu.VMEM((EPT,), jnp.int32),
        pltpu.VMEM((DEG_PAD,), jnp.float32),
        pltpu.SemaphoreType.DMA,
    ],
)


def _sc_hop_body(g_hbm, idx_hbm, zer_hbm, out_hbm,
                 idxb, rows0, rows1, acc,
                 gsem0, gsem1, ssem0, ssem1):
    cid = lax.axis_index("c")
    sid = lax.axis_index("s")
    wid = cid * NS + sid
    rows = (rows0, rows1)
    gsem = (gsem0, gsem1)
    ssem = (ssem0, ssem1)

    def _wait(b, sem):
        pltpu.make_async_copy(zer_hbm.at[pl.ds(0, CHUNK)], rows[b], sem[b]).wait()

    def _gather(j, b):
        pltpu.async_copy(g_hbm.at[idxb.at[0, j]], rows[b], gsem[b])

    def _scatter(j, b):
        pltpu.async_copy(rows[b], acc.at[idxb.at[1, j]], ssem[b], add=True)

    pltpu.sync_copy(zer_hbm.at[pl.ds(sid * RPT, RPT)], acc.at[pl.ds(sid * RPT, RPT)])

    @pl.when(sid == 0)
    def _():
        pltpu.sync_copy(zer_hbm.at[pl.ds(NS * RPT, ZREM)], acc.at[pl.ds(NS * RPT, ZREM)])

    plsc.subcore_barrier()

    def obody(o, carry):
        @pl.when(o > 0)
        def _():
            for b in range(NB):
                _wait(b, ssem)

        pltpu.sync_copy(idx_hbm.at[wid, :, pl.ds(o * IB, IB)], idxb)
        for b in range(NB):
            _gather(b, b)

        def ibody(p, c2):
            for b in range(NB):
                _wait(b, gsem)
                _scatter(NB * p + b, b)
            for b in range(NB):
                _wait(b, ssem)
                _gather(NB * p + NB + b, b)
            return c2

        lax.fori_loop(0, IB // NB - 1, ibody, 0)
        for b in range(NB):
            _wait(b, gsem)
            _scatter(IB - NB + b, b)
        return carry

    lax.fori_loop(0, OUTER, obody, 0)
    for b in range(NB):
        _wait(b, ssem)
    plsc.subcore_barrier()
    pltpu.sync_copy(acc.at[pl.ds(sid * RPT, RPT)], out_hbm.at[cid, pl.ds(sid * RPT, RPT)])

    @pl.when(sid == 0)
    def _():
        pltpu.sync_copy(acc.at[pl.ds(NS * RPT, OREM)], out_hbm.at[cid, pl.ds(NS * RPT, OREM)])


_sc_hop = pl.kernel(
    _sc_hop_body,
    out_type=jax.ShapeDtypeStruct((2, N, D), jnp.float32),
    mesh=_mesh,
    compiler_params=pltpu.CompilerParams(needs_layout_passes=False),
    scratch_types=[
        pltpu.VMEM((2, IB, CHUNK), jnp.int32),
        pltpu.VMEM((CHUNK, D), jnp.float32),
        pltpu.VMEM((CHUNK, D), jnp.float32),
        pltpu.VMEM_SHARED((N_PAD, D), jnp.float32),
        pltpu.SemaphoreType.DMA,
        pltpu.SemaphoreType.DMA,
        pltpu.SemaphoreType.DMA,
        pltpu.SemaphoreType.DMA,
    ],
)


def _gelu(x):
    return x * 0.5 * (1.0 + lax.erf(x * 0.7071067811865476))


def _tc_pre_body(x_ref, degp_ref, gamma_ref, beta_ref, w0_ref, b0_ref, wt0_ref,
                 xb_ref, g_ref, acc_ref, dis_ref):
    x = x_ref[...]
    mu = jnp.mean(x, axis=0, keepdims=True)
    xc = x - mu
    var = jnp.mean(xc * xc, axis=0, keepdims=True)
    xb = xc * lax.rsqrt(var + 1e-5) * gamma_ref[...] + beta_ref[...]
    xb_ref[...] = xb
    h = _gelu(jnp.dot(xb, w0_ref[...], preferred_element_type=jnp.float32) + b0_ref[...])
    deg = jnp.sum(degp_ref[...], axis=1, keepdims=True)[:N]
    dis = jnp.where(deg > 0, lax.rsqrt(jnp.maximum(deg, 1e-12)), 0.0)
    dis_ref[...] = dis
    g_ref[...] = dis * h
    acc_ref[...] = jnp.dot(h, wt0_ref[...], preferred_element_type=jnp.float32)


_tc_pre = pl.pallas_call(
    _tc_pre_body,
    out_shape=(
        jax.ShapeDtypeStruct((N, D), jnp.float32),
        jax.ShapeDtypeStruct((N, D), jnp.float32),
        jax.ShapeDtypeStruct((N, D), jnp.float32),
        jax.ShapeDtypeStruct((N, 1), jnp.float32),
    ),
)


def _tc_hop_body(s_ref, dis_ref, wt_ref, acc_in_ref, g_ref, acc_out_ref):
    s = s_ref[0] + s_ref[1]
    dis = dis_ref[...]
    cur = dis * s
    g_ref[...] = dis * cur
    acc_out_ref[...] = acc_in_ref[...] + jnp.dot(
        cur, wt_ref[...], preferred_element_type=jnp.float32)


_tc_hop = pl.pallas_call(
    _tc_hop_body,
    out_shape=(
        jax.ShapeDtypeStruct((N, D), jnp.float32),
        jax.ShapeDtypeStruct((N, D), jnp.float32),
    ),
)


def _tc_post_body(xb_ref, acc_ref, bt_ref, w1a_ref, w1b_ref, b1_ref,
                  w2_ref, b2_ref, wc_ref, bc_ref, we_ref, be_ref,
                  a_ref, cl_ref):
    h = _gelu(acc_ref[...] + bt_ref[...])
    a1 = _gelu(jnp.dot(xb_ref[...], w1a_ref[...], preferred_element_type=jnp.float32)
               + jnp.dot(h, w1b_ref[...], preferred_element_type=jnp.float32)
               + b1_ref[...])
    a2 = _gelu(jnp.dot(a1, w2_ref[...], preferred_element_type=jnp.float32) + b2_ref[...])
    a_ref[...] = a2
    cl = _gelu(jnp.dot(a2, wc_ref[...], preferred_element_type=jnp.float32) + bc_ref[...])
    logits = jnp.dot(cl, we_ref[...], preferred_element_type=jnp.float32) + be_ref[...]
    m = jnp.max(logits, axis=1, keepdims=True)
    e = jnp.exp(logits - m)
    cl_ref[...] = e / jnp.sum(e, axis=1, keepdims=True)


_tc_post = pl.pallas_call(
    _tc_post_body,
    out_shape=(
        jax.ShapeDtypeStruct((N, 64), jnp.float32),
        jax.ShapeDtypeStruct((N, 16), jnp.float32),
    ),
)


def kernel(x, edge_index, gamma, beta, W0, b0, Wt, bt, W1, b1, W2, b2, Wc, bc, We, be):
    row = edge_index[0]
    col = edge_index[1]
    pad = E_PAD - E
    rowp = jnp.concatenate([row, jnp.zeros((pad,), jnp.int32)]).reshape(NW, CPT, CHUNK)
    colp = jnp.concatenate([col, jnp.full((pad,), N, jnp.int32)]).reshape(NW, CPT, CHUNK)
    idxp = jnp.stack([rowp, colp], axis=1)
    degp = _sc_deg(colp.reshape(NW, EPT))
    zer = jnp.zeros((N_PAD, D), jnp.float32)
    xb, g, acc, dis = _tc_pre(x, degp.T, gamma.reshape(1, D), beta.reshape(1, D),
                              W0, b0.reshape(1, D), Wt[0])
    for k in range(1, K + 1):
        s = _sc_hop(g, idxp, zer)
        g, acc = _tc_hop(s, dis, Wt[k], acc)
    a, cl = _tc_post(xb, acc, bt.reshape(1, D), W1[:D], W1[D:], b1.reshape(1, 128),
                     W2, b2.reshape(1, 64), Wc, bc.reshape(1, 64),
                     We, be.reshape(1, 16))
    return (a, cl)


# speedup vs baseline: 1.1577x; 1.0261x over previous
import functools

import jax
import jax.numpy as jnp
from jax import lax
from jax.experimental import pallas as pl
from jax.experimental.pallas import tpu as pltpu
from jax.experimental.pallas import tpu_sc as plsc

N = 10000
D = 128
E = 640000
K = 6

NW = 32
NS = 16
CHUNK = 128
NB = 2
IB = 32
OUTER = 5
CPT = IB * OUTER
EPT = CPT * CHUNK
E_PAD = NW * EPT
N_PAD = 10008
DEG_PAD = 10016
RPT = 624
ZREM = N_PAD - NS * RPT
OREM = N - NS * RPT

_mesh = plsc.VectorSubcoreMesh(core_axis_name="c", subcore_axis_name="s")


def _sc_deg_body(col_hbm, out_hbm, colv, degv, sem):
    cid = lax.axis_index("c")
    sid = lax.axis_index("s")
    wid = cid * NS + sid
    pltpu.async_copy(col_hbm.at[wid], colv, sem).wait()
    zeros16 = jnp.zeros((16,), jnp.float32)

    def zbody(i, carry):
        degv[pl.ds(i * 16, 16)] = zeros16
        return carry

    lax.fori_loop(0, DEG_PAD // 16, zbody, 0)
    ones16 = jnp.ones((16,), jnp.float32)

    def ebody(j, carry):
        idx = colv[pl.ds(j * 16, 16)]
        plsc.addupdate_scatter(degv, [idx], ones16)
        return carry

    lax.fori_loop(0, EPT // 16, ebody, 0)
    pltpu.sync_copy(degv, out_hbm.at[wid])


_sc_deg = pl.kernel(
    _sc_deg_body,
    out_type=jax.ShapeDtypeStruct((NW, DEG_PAD), jnp.float32),
    mesh=_mesh,
    compiler_params=pltpu.CompilerParams(needs_layout_passes=False),
    scratch_types=[
        pltpu.VMEM((EPT,), jnp.int32),
        pltpu.VMEM((DEG_PAD,), jnp.float32),
        pltpu.SemaphoreType.DMA,
    ],
)


def _sc_hop_body(g_hbm, idx_hbm, zer_hbm, out_hbm,
                 idxb, rows0, rows1, acc,
                 gsem0, gsem1, ssem0, ssem1):
    cid = lax.axis_index("c")
    sid = lax.axis_index("s")
    wid = cid * NS + sid
    rows = (rows0, rows1)
    gsem = (gsem0, gsem1)
    ssem = (ssem0, ssem1)

    def _wait(b, sem):
        pltpu.make_async_copy(zer_hbm.at[pl.ds(0, CHUNK)], rows[b], sem[b]).wait()

    def _gather(j, b):
        pltpu.async_copy(g_hbm.at[idxb.at[0, j]], rows[b], gsem[b])

    def _scatter(j, b):
        pltpu.sync_copy(rows[b], acc.at[idxb.at[1, j]], add=True)

    pltpu.sync_copy(zer_hbm.at[pl.ds(sid * RPT, RPT)], acc.at[pl.ds(sid * RPT, RPT)])

    @pl.when(sid == 0)
    def _():
        pltpu.sync_copy(zer_hbm.at[pl.ds(NS * RPT, ZREM)], acc.at[pl.ds(NS * RPT, ZREM)])

    plsc.subcore_barrier()

    def obody(o, carry):
        pltpu.sync_copy(idx_hbm.at[wid, :, pl.ds(o * IB, IB)], idxb)
        _gather(0, 0)

        def ibody(p, c2):
            _gather(2 * p + 1, 1)
            _wait(0, gsem)
            _scatter(2 * p, 0)
            _gather(2 * p + 2, 0)
            _wait(1, gsem)
            _scatter(2 * p + 1, 1)
            return c2

        lax.fori_loop(0, IB // 2 - 1, ibody, 0)
        _gather(IB - 1, 1)
        _wait(0, gsem)
        _scatter(IB - 2, 0)
        _wait(1, gsem)
        _scatter(IB - 1, 1)
        return carry

    lax.fori_loop(0, OUTER, obody, 0)
    plsc.subcore_barrier()
    pltpu.sync_copy(acc.at[pl.ds(sid * RPT, RPT)], out_hbm.at[cid, pl.ds(sid * RPT, RPT)])

    @pl.when(sid == 0)
    def _():
        pltpu.sync_copy(acc.at[pl.ds(NS * RPT, OREM)], out_hbm.at[cid, pl.ds(NS * RPT, OREM)])


_sc_hop = pl.kernel(
    _sc_hop_body,
    out_type=jax.ShapeDtypeStruct((2, N, D), jnp.float32),
    mesh=_mesh,
    compiler_params=pltpu.CompilerParams(needs_layout_passes=False),
    scratch_types=[
        pltpu.VMEM((2, IB, CHUNK), jnp.int32),
        pltpu.VMEM((CHUNK, D), jnp.float32),
        pltpu.VMEM((CHUNK, D), jnp.float32),
        pltpu.VMEM_SHARED((N_PAD, D), jnp.float32),
        pltpu.SemaphoreType.DMA,
        pltpu.SemaphoreType.DMA,
        pltpu.SemaphoreType.DMA,
        pltpu.SemaphoreType.DMA,
    ],
)


def _gelu(x):
    return x * 0.5 * (1.0 + lax.erf(x * 0.7071067811865476))


def _tc_pre_body(x_ref, degp_ref, gamma_ref, beta_ref, w0_ref, b0_ref, wt0_ref,
                 xb_ref, g_ref, acc_ref, dis_ref):
    x = x_ref[...]
    mu = jnp.mean(x, axis=0, keepdims=True)
    xc = x - mu
    var = jnp.mean(xc * xc, axis=0, keepdims=True)
    xb = xc * lax.rsqrt(var + 1e-5) * gamma_ref[...] + beta_ref[...]
    xb_ref[...] = xb
    h = _gelu(jnp.dot(xb, w0_ref[...], preferred_element_type=jnp.float32) + b0_ref[...])
    deg = jnp.sum(degp_ref[...], axis=1, keepdims=True)[:N]
    dis = jnp.where(deg > 0, lax.rsqrt(jnp.maximum(deg, 1e-12)), 0.0)
    dis_ref[...] = dis
    g_ref[...] = dis * h
    acc_ref[...] = jnp.dot(h, wt0_ref[...], preferred_element_type=jnp.float32)


_tc_pre = pl.pallas_call(
    _tc_pre_body,
    out_shape=(
        jax.ShapeDtypeStruct((N, D), jnp.float32),
        jax.ShapeDtypeStruct((N, D), jnp.float32),
        jax.ShapeDtypeStruct((N, D), jnp.float32),
        jax.ShapeDtypeStruct((N, 1), jnp.float32),
    ),
)


def _tc_hop_body(s_ref, dis_ref, wt_ref, acc_in_ref, g_ref, acc_out_ref):
    s = s_ref[0] + s_ref[1]
    dis = dis_ref[...]
    cur = dis * s
    g_ref[...] = dis * cur
    acc_out_ref[...] = acc_in_ref[...] + jnp.dot(
        cur, wt_ref[...], preferred_element_type=jnp.float32)


_tc_hop = pl.pallas_call(
    _tc_hop_body,
    out_shape=(
        jax.ShapeDtypeStruct((N, D), jnp.float32),
        jax.ShapeDtypeStruct((N, D), jnp.float32),
    ),
)


def _tc_post_body(xb_ref, acc_ref, bt_ref, w1a_ref, w1b_ref, b1_ref,
                  w2_ref, b2_ref, wc_ref, bc_ref, we_ref, be_ref,
                  a_ref, cl_ref):
    h = _gelu(acc_ref[...] + bt_ref[...])
    a1 = _gelu(jnp.dot(xb_ref[...], w1a_ref[...], preferred_element_type=jnp.float32)
               + jnp.dot(h, w1b_ref[...], preferred_element_type=jnp.float32)
               + b1_ref[...])
    a2 = _gelu(jnp.dot(a1, w2_ref[...], preferred_element_type=jnp.float32) + b2_ref[...])
    a_ref[...] = a2
    cl = _gelu(jnp.dot(a2, wc_ref[...], preferred_element_type=jnp.float32) + bc_ref[...])
    logits = jnp.dot(cl, we_ref[...], preferred_element_type=jnp.float32) + be_ref[...]
    m = jnp.max(logits, axis=1, keepdims=True)
    e = jnp.exp(logits - m)
    cl_ref[...] = e / jnp.sum(e, axis=1, keepdims=True)


_tc_post = pl.pallas_call(
    _tc_post_body,
    out_shape=(
        jax.ShapeDtypeStruct((N, 64), jnp.float32),
        jax.ShapeDtypeStruct((N, 16), jnp.float32),
    ),
)


def kernel(x, edge_index, gamma, beta, W0, b0, Wt, bt, W1, b1, W2, b2, Wc, bc, We, be):
    row = edge_index[0]
    col = edge_index[1]
    pad = E_PAD - E
    rowp = jnp.concatenate([row, jnp.zeros((pad,), jnp.int32)]).reshape(NW, CPT, CHUNK)
    colp = jnp.concatenate([col, jnp.full((pad,), N, jnp.int32)]).reshape(NW, CPT, CHUNK)
    idxp = jnp.stack([rowp, colp], axis=1)
    degp = _sc_deg(colp.reshape(NW, EPT))
    zer = jnp.zeros((N_PAD, D), jnp.float32)
    xb, g, acc, dis = _tc_pre(x, degp.T, gamma.reshape(1, D), beta.reshape(1, D),
                              W0, b0.reshape(1, D), Wt[0])
    for k in range(1, K + 1):
        s = _sc_hop(g, idxp, zer)
        g, acc = _tc_hop(s, dis, Wt[k], acc)
    a, cl = _tc_post(xb, acc, bt.reshape(1, D), W1[:D], W1[D:], b1.reshape(1, 128),
                     W2, b2.reshape(1, 64), Wc, bc.reshape(1, 64),
                     We, be.reshape(1, 16))
    return (a, cl)


# speedup vs baseline: 1.3535x; 1.1692x over previous
import functools

import jax
import jax.numpy as jnp
from jax import lax
from jax.experimental import pallas as pl
from jax.experimental.pallas import tpu as pltpu
from jax.experimental.pallas import tpu_sc as plsc

N = 10000
D = 128
E = 640000
K = 6

NW = 32
NS = 16
CHUNK = 128
NB = 2
IB = 32
OUTER = 5
CPT = IB * OUTER
EPT = CPT * CHUNK
E_PAD = NW * EPT
N_PAD = 10008
DEG_PAD = 10016
RPT = 624
ZREM = N_PAD - NS * RPT
OREM = N - NS * RPT

_mesh = plsc.VectorSubcoreMesh(core_axis_name="c", subcore_axis_name="s")


def _sc_deg_body(col_hbm, out_hbm, colv, degv, sem):
    cid = lax.axis_index("c")
    sid = lax.axis_index("s")
    wid = cid * NS + sid
    pltpu.async_copy(col_hbm.at[wid], colv, sem).wait()
    zeros16 = jnp.zeros((16,), jnp.float32)

    def zbody(i, carry):
        degv[pl.ds(i * 16, 16)] = zeros16
        return carry

    lax.fori_loop(0, DEG_PAD // 16, zbody, 0)
    ones16 = jnp.ones((16,), jnp.float32)

    def ebody(j, carry):
        idx = colv[pl.ds(j * 16, 16)]
        plsc.addupdate_scatter(degv, [idx], ones16)
        return carry

    lax.fori_loop(0, EPT // 16, ebody, 0)
    pltpu.sync_copy(degv, out_hbm.at[wid])


_sc_deg = pl.kernel(
    _sc_deg_body,
    out_type=jax.ShapeDtypeStruct((NW, DEG_PAD), jnp.float32),
    mesh=_mesh,
    compiler_params=pltpu.CompilerParams(needs_layout_passes=False),
    scratch_types=[
        pltpu.VMEM((EPT,), jnp.int32),
        pltpu.VMEM((DEG_PAD,), jnp.float32),
        pltpu.SemaphoreType.DMA,
    ],
)


def _sc_hop_body(g_hbm, row_hbm, col_hbm, zer_hbm, out_hbm,
                 rowb, colb, rows0, rows1, acc,
                 gsem0, gsem1, ssem0, ssem1):
    cid = lax.axis_index("c")
    sid = lax.axis_index("s")
    wid = cid * NS + sid
    rows = (rows0, rows1)
    gsem = (gsem0, gsem1)
    ssem = (ssem0, ssem1)

    def _wait(b, sem):
        pltpu.make_async_copy(zer_hbm.at[pl.ds(0, CHUNK)], rows[b], sem[b]).wait()

    def _gather(j, b):
        pltpu.async_copy(g_hbm.at[rowb.at[j]], rows[b], gsem[b])

    def _scatter(j, b):
        pltpu.sync_copy(rows[b], acc.at[colb.at[j]], add=True)

    pltpu.sync_copy(zer_hbm.at[pl.ds(sid * RPT, RPT)], acc.at[pl.ds(sid * RPT, RPT)])

    @pl.when(sid == 0)
    def _():
        pltpu.sync_copy(zer_hbm.at[pl.ds(NS * RPT, ZREM)], acc.at[pl.ds(NS * RPT, ZREM)])

    plsc.subcore_barrier()

    def obody(o, carry):
        pltpu.sync_copy(row_hbm.at[wid, pl.ds(o * IB, IB)], rowb)
        pltpu.sync_copy(col_hbm.at[wid, pl.ds(o * IB, IB)], colb)
        _gather(0, 0)

        def ibody(p, c2):
            _gather(2 * p + 1, 1)
            _wait(0, gsem)
            _scatter(2 * p, 0)
            _gather(2 * p + 2, 0)
            _wait(1, gsem)
            _scatter(2 * p + 1, 1)
            return c2

        lax.fori_loop(0, IB // 2 - 1, ibody, 0)
        _gather(IB - 1, 1)
        _wait(0, gsem)
        _scatter(IB - 2, 0)
        _wait(1, gsem)
        _scatter(IB - 1, 1)
        return carry

    lax.fori_loop(0, OUTER, obody, 0)
    plsc.subcore_barrier()
    pltpu.sync_copy(acc.at[pl.ds(sid * RPT, RPT)], out_hbm.at[cid, pl.ds(sid * RPT, RPT)])

    @pl.when(sid == 0)
    def _():
        pltpu.sync_copy(acc.at[pl.ds(NS * RPT, OREM)], out_hbm.at[cid, pl.ds(NS * RPT, OREM)])


_sc_hop = pl.kernel(
    _sc_hop_body,
    out_type=jax.ShapeDtypeStruct((2, N, D), jnp.float32),
    mesh=_mesh,
    compiler_params=pltpu.CompilerParams(needs_layout_passes=False),
    scratch_types=[
        pltpu.VMEM((IB, CHUNK), jnp.int32),
        pltpu.VMEM((IB, CHUNK), jnp.int32),
        pltpu.VMEM((CHUNK, D), jnp.float32),
        pltpu.VMEM((CHUNK, D), jnp.float32),
        pltpu.VMEM_SHARED((N_PAD, D), jnp.float32),
        pltpu.SemaphoreType.DMA,
        pltpu.SemaphoreType.DMA,
        pltpu.SemaphoreType.DMA,
        pltpu.SemaphoreType.DMA,
    ],
)


def _gelu(x):
    return x * 0.5 * (1.0 + lax.erf(x * 0.7071067811865476))


def _tc_pre_body(x_ref, degp_ref, gamma_ref, beta_ref, w0_ref, b0_ref, wt0_ref,
                 xb_ref, g_ref, acc_ref, dis_ref):
    x = x_ref[...]
    mu = jnp.mean(x, axis=0, keepdims=True)
    xc = x - mu
    var = jnp.mean(xc * xc, axis=0, keepdims=True)
    xb = xc * lax.rsqrt(var + 1e-5) * gamma_ref[...] + beta_ref[...]
    xb_ref[...] = xb
    h = _gelu(jnp.dot(xb, w0_ref[...], preferred_element_type=jnp.float32) + b0_ref[...])
    deg = jnp.sum(degp_ref[...], axis=1, keepdims=True)[:N]
    dis = jnp.where(deg > 0, lax.rsqrt(jnp.maximum(deg, 1e-12)), 0.0)
    dis_ref[...] = dis
    g_ref[...] = dis * h
    acc_ref[...] = jnp.dot(h, wt0_ref[...], preferred_element_type=jnp.float32)


_tc_pre = pl.pallas_call(
    _tc_pre_body,
    out_shape=(
        jax.ShapeDtypeStruct((N, D), jnp.float32),
        jax.ShapeDtypeStruct((N, D), jnp.float32),
        jax.ShapeDtypeStruct((N, D), jnp.float32),
        jax.ShapeDtypeStruct((N, 1), jnp.float32),
    ),
)


def _tc_hop_body(s_ref, dis_ref, wt_ref, acc_in_ref, g_ref, acc_out_ref):
    s = s_ref[0] + s_ref[1]
    dis = dis_ref[...]
    cur = dis * s
    g_ref[...] = dis * cur
    acc_out_ref[...] = acc_in_ref[...] + jnp.dot(
        cur, wt_ref[...], preferred_element_type=jnp.float32)


_tc_hop = pl.pallas_call(
    _tc_hop_body,
    out_shape=(
        jax.ShapeDtypeStruct((N, D), jnp.float32),
        jax.ShapeDtypeStruct((N, D), jnp.float32),
    ),
)


def _tc_post_body(xb_ref, acc_ref, bt_ref, w1a_ref, w1b_ref, b1_ref,
                  w2_ref, b2_ref, wc_ref, bc_ref, we_ref, be_ref,
                  a_ref, cl_ref):
    h = _gelu(acc_ref[...] + bt_ref[...])
    a1 = _gelu(jnp.dot(xb_ref[...], w1a_ref[...], preferred_element_type=jnp.float32)
               + jnp.dot(h, w1b_ref[...], preferred_element_type=jnp.float32)
               + b1_ref[...])
    a2 = _gelu(jnp.dot(a1, w2_ref[...], preferred_element_type=jnp.float32) + b2_ref[...])
    a_ref[...] = a2
    cl = _gelu(jnp.dot(a2, wc_ref[...], preferred_element_type=jnp.float32) + bc_ref[...])
    logits = jnp.dot(cl, we_ref[...], preferred_element_type=jnp.float32) + be_ref[...]
    m = jnp.max(logits, axis=1, keepdims=True)
    e = jnp.exp(logits - m)
    cl_ref[...] = e / jnp.sum(e, axis=1, keepdims=True)


_tc_post = pl.pallas_call(
    _tc_post_body,
    out_shape=(
        jax.ShapeDtypeStruct((N, 64), jnp.float32),
        jax.ShapeDtypeStruct((N, 16), jnp.float32),
    ),
)


def kernel(x, edge_index, gamma, beta, W0, b0, Wt, bt, W1, b1, W2, b2, Wc, bc, We, be):
    row = edge_index[0]
    col = edge_index[1]
    pad = E_PAD - E
    rowp = jnp.concatenate([row, jnp.zeros((pad,), jnp.int32)]).reshape(NW, CPT, CHUNK)
    colp = jnp.concatenate([col, jnp.full((pad,), N, jnp.int32)]).reshape(NW, CPT, CHUNK)
    degp = _sc_deg(colp.reshape(NW, EPT))
    zer = jnp.zeros((N_PAD, D), jnp.float32)
    xb, g, acc, dis = _tc_pre(x, degp.T, gamma.reshape(1, D), beta.reshape(1, D),
                              W0, b0.reshape(1, D), Wt[0])
    for k in range(1, K + 1):
        s = _sc_hop(g, rowp, colp, zer)
        g, acc = _tc_hop(s, dis, Wt[k], acc)
    a, cl = _tc_post(xb, acc, bt.reshape(1, D), W1[:D], W1[D:], b1.reshape(1, 128),
                     W2, b2.reshape(1, 64), Wc, bc.reshape(1, 64),
                     We, be.reshape(1, 16))
    return (a, cl)


# speedup vs baseline: 1.3590x; 1.0040x over previous
import functools

import jax
import jax.numpy as jnp
from jax import lax
from jax.experimental import pallas as pl
from jax.experimental.pallas import tpu as pltpu
from jax.experimental.pallas import tpu_sc as plsc

N = 10000
D = 128
E = 640000
K = 6

NW = 32
NS = 16
CHUNK = 128
NB = 2
IB = 40
OUTER = 4
CPT = IB * OUTER
EPT = CPT * CHUNK
E_PAD = NW * EPT
N_PAD = 10008
DEG_PAD = 10016
RPT = 624
ZREM = N_PAD - NS * RPT
OREM = N - NS * RPT

_mesh = plsc.VectorSubcoreMesh(core_axis_name="c", subcore_axis_name="s")


def _sc_deg_body(col_hbm, out_hbm, colv, degv, sem):
    cid = lax.axis_index("c")
    sid = lax.axis_index("s")
    wid = cid * NS + sid
    pltpu.async_copy(col_hbm.at[wid], colv, sem).wait()
    zeros16 = jnp.zeros((16,), jnp.float32)

    def zbody(i, carry):
        degv[pl.ds(i * 16, 16)] = zeros16
        return carry

    lax.fori_loop(0, DEG_PAD // 16, zbody, 0)
    ones16 = jnp.ones((16,), jnp.float32)

    def ebody(j, carry):
        idx = colv[pl.ds(j * 16, 16)]
        plsc.addupdate_scatter(degv, [idx], ones16)
        return carry

    lax.fori_loop(0, EPT // 16, ebody, 0)
    pltpu.sync_copy(degv, out_hbm.at[wid])


_sc_deg = pl.kernel(
    _sc_deg_body,
    out_type=jax.ShapeDtypeStruct((NW, DEG_PAD), jnp.float32),
    mesh=_mesh,
    compiler_params=pltpu.CompilerParams(needs_layout_passes=False),
    scratch_types=[
        pltpu.VMEM((EPT,), jnp.int32),
        pltpu.VMEM((DEG_PAD,), jnp.float32),
        pltpu.SemaphoreType.DMA,
    ],
)


def _sc_hop_body(g_hbm, row_hbm, col_hbm, zer_hbm, out_hbm,
                 rowb, colb, rows0, rows1, acc,
                 gsem0, gsem1, ssem0, ssem1):
    cid = lax.axis_index("c")
    sid = lax.axis_index("s")
    wid = cid * NS + sid
    rows = (rows0, rows1)
    gsem = (gsem0, gsem1)
    ssem = (ssem0, ssem1)

    def _wait(b, sem):
        pltpu.make_async_copy(zer_hbm.at[pl.ds(0, CHUNK)], rows[b], sem[b]).wait()

    def _gather(j, b):
        pltpu.async_copy(g_hbm.at[rowb.at[j]], rows[b], gsem[b])

    def _scatter(j, b):
        pltpu.sync_copy(rows[b], acc.at[colb.at[j]], add=True)

    pltpu.sync_copy(zer_hbm.at[pl.ds(sid * RPT, RPT)], acc.at[pl.ds(sid * RPT, RPT)])

    @pl.when(sid == 0)
    def _():
        pltpu.sync_copy(zer_hbm.at[pl.ds(NS * RPT, ZREM)], acc.at[pl.ds(NS * RPT, ZREM)])

    plsc.subcore_barrier()

    def obody(o, carry):
        pltpu.sync_copy(row_hbm.at[wid, pl.ds(o * IB, IB)], rowb)
        pltpu.sync_copy(col_hbm.at[wid, pl.ds(o * IB, IB)], colb)
        _gather(0, 0)

        def ibody(p, c2):
            _gather(2 * p + 1, 1)
            _wait(0, gsem)
            _scatter(2 * p, 0)
            _gather(2 * p + 2, 0)
            _wait(1, gsem)
            _scatter(2 * p + 1, 1)
            return c2

        lax.fori_loop(0, IB // 2 - 1, ibody, 0)
        _gather(IB - 1, 1)
        _wait(0, gsem)
        _scatter(IB - 2, 0)
        _wait(1, gsem)
        _scatter(IB - 1, 1)
        return carry

    lax.fori_loop(0, OUTER, obody, 0)
    plsc.subcore_barrier()
    pltpu.sync_copy(acc.at[pl.ds(sid * RPT, RPT)], out_hbm.at[cid, pl.ds(sid * RPT, RPT)])

    @pl.when(sid == 0)
    def _():
        pltpu.sync_copy(acc.at[pl.ds(NS * RPT, OREM)], out_hbm.at[cid, pl.ds(NS * RPT, OREM)])


_sc_hop = pl.kernel(
    _sc_hop_body,
    out_type=jax.ShapeDtypeStruct((2, N, D), jnp.float32),
    mesh=_mesh,
    compiler_params=pltpu.CompilerParams(needs_layout_passes=False),
    scratch_types=[
        pltpu.VMEM((IB, CHUNK), jnp.int32),
        pltpu.VMEM((IB, CHUNK), jnp.int32),
        pltpu.VMEM((CHUNK, D), jnp.float32),
        pltpu.VMEM((CHUNK, D), jnp.float32),
        pltpu.VMEM_SHARED((N_PAD, D), jnp.float32),
        pltpu.SemaphoreType.DMA,
        pltpu.SemaphoreType.DMA,
        pltpu.SemaphoreType.DMA,
        pltpu.SemaphoreType.DMA,
    ],
)


def _gelu(x):
    return x * 0.5 * (1.0 + lax.erf(x * 0.7071067811865476))


def _tc_pre_body(x_ref, degp_ref, gamma_ref, beta_ref, w0_ref, b0_ref, wt0_ref,
                 xb_ref, g_ref, acc_ref, dis_ref):
    x = x_ref[...]
    mu = jnp.mean(x, axis=0, keepdims=True)
    xc = x - mu
    var = jnp.mean(xc * xc, axis=0, keepdims=True)
    xb = xc * lax.rsqrt(var + 1e-5) * gamma_ref[...] + beta_ref[...]
    xb_ref[...] = xb
    h = _gelu(jnp.dot(xb, w0_ref[...], preferred_element_type=jnp.float32) + b0_ref[...])
    deg = jnp.sum(degp_ref[...], axis=1, keepdims=True)[:N]
    dis = jnp.where(deg > 0, lax.rsqrt(jnp.maximum(deg, 1e-12)), 0.0)
    dis_ref[...] = dis
    g_ref[...] = dis * h
    acc_ref[...] = jnp.dot(h, wt0_ref[...], preferred_element_type=jnp.float32)


_tc_pre = pl.pallas_call(
    _tc_pre_body,
    out_shape=(
        jax.ShapeDtypeStruct((N, D), jnp.float32),
        jax.ShapeDtypeStruct((N, D), jnp.float32),
        jax.ShapeDtypeStruct((N, D), jnp.float32),
        jax.ShapeDtypeStruct((N, 1), jnp.float32),
    ),
)


def _tc_hop_body(s_ref, dis_ref, wt_ref, acc_in_ref, g_ref, acc_out_ref):
    s = s_ref[0] + s_ref[1]
    dis = dis_ref[...]
    cur = dis * s
    g_ref[...] = dis * cur
    acc_out_ref[...] = acc_in_ref[...] + jnp.dot(
        cur, wt_ref[...], preferred_element_type=jnp.float32)


_tc_hop = pl.pallas_call(
    _tc_hop_body,
    out_shape=(
        jax.ShapeDtypeStruct((N, D), jnp.float32),
        jax.ShapeDtypeStruct((N, D), jnp.float32),
    ),
)


def _tc_post_body(xb_ref, acc_ref, bt_ref, w1a_ref, w1b_ref, b1_ref,
                  w2_ref, b2_ref, wc_ref, bc_ref, we_ref, be_ref,
                  a_ref, cl_ref):
    h = _gelu(acc_ref[...] + bt_ref[...])
    a1 = _gelu(jnp.dot(xb_ref[...], w1a_ref[...], preferred_element_type=jnp.float32)
               + jnp.dot(h, w1b_ref[...], preferred_element_type=jnp.float32)
               + b1_ref[...])
    a2 = _gelu(jnp.dot(a1, w2_ref[...], preferred_element_type=jnp.float32) + b2_ref[...])
    a_ref[...] = a2
    cl = _gelu(jnp.dot(a2, wc_ref[...], preferred_element_type=jnp.float32) + bc_ref[...])
    logits = jnp.dot(cl, we_ref[...], preferred_element_type=jnp.float32) + be_ref[...]
    m = jnp.max(logits, axis=1, keepdims=True)
    e = jnp.exp(logits - m)
    cl_ref[...] = e / jnp.sum(e, axis=1, keepdims=True)


_tc_post = pl.pallas_call(
    _tc_post_body,
    out_shape=(
        jax.ShapeDtypeStruct((N, 64), jnp.float32),
        jax.ShapeDtypeStruct((N, 16), jnp.float32),
    ),
)


def kernel(x, edge_index, gamma, beta, W0, b0, Wt, bt, W1, b1, W2, b2, Wc, bc, We, be):
    row = edge_index[0]
    col = edge_index[1]
    pad = E_PAD - E
    rowp = jnp.concatenate([row, jnp.zeros((pad,), jnp.int32)]).reshape(NW, CPT, CHUNK)
    colp = jnp.concatenate([col, jnp.full((pad,), N, jnp.int32)]).reshape(NW, CPT, CHUNK)
    degp = _sc_deg(colp.reshape(NW, EPT))
    zer = jnp.zeros((N_PAD, D), jnp.float32)
    xb, g, acc, dis = _tc_pre(x, degp.T, gamma.reshape(1, D), beta.reshape(1, D),
                              W0, b0.reshape(1, D), Wt[0])
    for k in range(1, K + 1):
        s = _sc_hop(g, rowp, colp, zer)
        g, acc = _tc_hop(s, dis, Wt[k], acc)
    a, cl = _tc_post(xb, acc, bt.reshape(1, D), W1[:D], W1[D:], b1.reshape(1, 128),
                     W2, b2.reshape(1, 64), Wc, bc.reshape(1, 64),
                     We, be.reshape(1, 16))
    return (a, cl)
